# baseline (speedup 1.0000x reference)
import dataclasses
import functools

import jax
import jax.numpy as jnp
from jax import lax
from jax.experimental import pallas as pl
from jax.experimental.pallas import tpu as pltpu
from jax.experimental.pallas import tpu_sc as plsc

NC = 2
NS = 16
L = 16
NW = NC * NS
CHUNK = 16


def _dense_pre(user, item, w_user, w_item, w0):
    nu = user.shape[0]
    ni = item.shape[0]
    d = w0.shape[1]

    def body(u_ref, i_ref, wu_ref, wi_ref, w0_ref, o_ref):
        wu0 = jnp.dot(wu_ref[...], w0_ref[...], preferred_element_type=jnp.float32)
        wi0 = jnp.dot(wi_ref[...], w0_ref[...], preferred_element_type=jnp.float32)
        o_ref[:nu, :] = jnp.dot(u_ref[...], wu0, preferred_element_type=jnp.float32)
        o_ref[nu:, :] = jnp.dot(i_ref[...], wi0, preferred_element_type=jnp.float32)

    return pl.pallas_call(
        body,
        out_shape=jax.ShapeDtypeStruct((nu + ni, d), jnp.float32),
    )(user, item, w_user, w_item, w0)


def _dense_mid(h2, w1, n):
    npad = h2.shape[0] // 2
    d = w1.shape[1]

    def body(h_ref, w_ref, o_ref):
        h = jnp.maximum(h_ref[:n, :] + h_ref[npad : npad + n, :], 0.0)
        o_ref[...] = jnp.dot(h, w_ref[...], preferred_element_type=jnp.float32)

    return pl.pallas_call(
        body,
        out_shape=jax.ShapeDtypeStruct((n, d), jnp.float32),
    )(h2, w1)


def _combine_split(g2, n, n_user):
    npad = g2.shape[0] // 2
    d = g2.shape[1]

    def body(g_ref, u_ref, i_ref):
        u_ref[...] = g_ref[:n_user, :] + g_ref[npad : npad + n_user, :]
        i_ref[...] = g_ref[n_user:n, :] + g_ref[npad + n_user : npad + n, :]

    return pl.pallas_call(
        body,
        out_shape=(
            jax.ShapeDtypeStruct((n_user, d), jnp.float32),
            jax.ShapeDtypeStruct((n - n_user, d), jnp.float32),
        ),
    )(g2)


def _spmm_sc(x, rows2, cols2, vals2):
    n, d = x.shape
    nchunk = rows2.shape[0] // (NW * CHUNK)
    per_w = nchunk * CHUNK
    rpt = (-(-n // NS) + 7) // 8 * 8
    npad = rpt * NS

    mesh = plsc.VectorSubcoreMesh(core_axis_name="c", subcore_axis_name="s")
    cp = pltpu.CompilerParams()
    if "needs_layout_passes" in pltpu.CompilerParams.__dataclass_fields__:
        cp = dataclasses.replace(cp, needs_layout_passes=False)

    @functools.partial(
        pl.kernel,
        out_type=jax.ShapeDtypeStruct((NC * npad, d), jnp.float32),
        mesh=mesh,
        compiler_params=cp,
        scratch_types=[
            pltpu.VMEM_SHARED((npad, d), jnp.float32),
            pltpu.VMEM((nchunk * CHUNK,), jnp.int32),
            pltpu.VMEM((nchunk * CHUNK,), jnp.float32),
            pltpu.VMEM((CHUNK, d), jnp.float32),
            pltpu.VMEM((CHUNK, d), jnp.float32),
            pltpu.VMEM((CHUNK,), jnp.int32),
            pltpu.VMEM((CHUNK,), jnp.int32),
            pltpu.SemaphoreType.DMA,
            pltpu.SemaphoreType.DMA,
            pltpu.SemaphoreType.DMA,
        ],
    )
    def run(x_hbm, rows_hbm, cols_hbm, vals_hbm, zeros_hbm, out_hbm,
            acc, cols_loc, vals_loc, rbuf0, rbuf1, ridx0, ridx1,
            gsem0, gsem1, zsem):
        ci = lax.axis_index("c")
        si = lax.axis_index("s")
        wid = ci * NS + si
        rbase = wid * per_w
        zcopy = pltpu.async_copy(zeros_hbm.at[pl.ds(si * rpt, rpt)],
                                 acc.at[pl.ds(si * rpt, rpt)], zsem)
        pltpu.sync_copy(cols_hbm.at[wid], cols_loc)
        pltpu.sync_copy(vals_hbm.at[wid], vals_loc)
        zcopy.wait()
        plsc.subcore_barrier()

        def refs(q, rbuf, ridx, gsem):
            g = pltpu.make_async_copy(
                x_hbm.at[cols_loc.at[pl.ds(q * CHUNK, CHUNK)]], rbuf, gsem)
            r = pltpu.make_async_copy(
                rows_hbm.at[pl.ds(rbase + q * CHUNK, CHUNK)], ridx, gsem)
            return g, r

        def issue_gather(q, rbuf, ridx, gsem):
            pltpu.async_copy(
                x_hbm.at[cols_loc.at[pl.ds(q * CHUNK, CHUNK)]], rbuf, gsem)
            pltpu.async_copy(
                rows_hbm.at[pl.ds(rbase + q * CHUNK, CHUNK)], ridx, gsem)

        issue_gather(0, rbuf0, ridx0, gsem0)
        issue_gather(1, rbuf1, ridx1, gsem1)

        def half(q, rbuf, ridx, gsem):
            g, r = refs(q, rbuf, ridx, gsem)
            g.wait()
            r.wait()
            @pl.loop(0, CHUNK // L)
            def _grp(g):
                e0 = g * L
                for j in range(L):
                    idx = jnp.broadcast_to(q * CHUNK + e0 + j, (L,))
                    v = plsc.load_gather(vals_loc, [idx.astype(jnp.int32)])
                    for f in range(0, d, L):
                        rbuf[e0 + j, pl.ds(f, L)] = rbuf[e0 + j, pl.ds(f, L)] * v
            pltpu.sync_copy(rbuf, acc.at[ridx], add=True)

            @pl.when(q < nchunk - 2)
            def _():
                issue_gather(q + 2, rbuf, ridx, gsem)

        @pl.loop(0, nchunk, step=2)
        def _pair(k):
            half(k, rbuf0, ridx0, gsem0)
            half(k + 1, rbuf1, ridx1, gsem1)

        plsc.subcore_barrier()
        pltpu.sync_copy(acc.at[pl.ds(si * rpt, rpt)],
                        out_hbm.at[pl.ds(ci * npad + si * rpt, rpt)])

    zeros = jnp.zeros((npad, d), jnp.float32)
    return run(x, rows2, cols2, vals2, zeros)


def kernel(user_attrs_input, item_attrs_input, edge_index, adj_vals,
           W_user, W_item, W0, W1):
    n_user = user_attrs_input.shape[0]
    e = edge_index.shape[1]

    per_w = -(-e // (NW * 2 * CHUNK)) * 2 * CHUNK
    pad = NW * per_w - e
    rows = edge_index[0].astype(jnp.int32)
    cols = edge_index[1].astype(jnp.int32)
    vals = adj_vals.astype(jnp.float32)
    if pad:
        rows = jnp.concatenate([rows, jnp.zeros((pad,), jnp.int32)])
        cols = jnp.concatenate([cols, jnp.zeros((pad,), jnp.int32)])
        vals = jnp.concatenate([vals, jnp.zeros((pad,), jnp.float32)])
    rows2 = rows
    cols2 = cols.reshape(NW, per_w)
    vals2 = vals.reshape(NW, per_w)

    x0 = _dense_pre(user_attrs_input, item_attrs_input, W_user, W_item, W0)
    n = x0.shape[0]
    h2 = _spmm_sc(x0, rows2, cols2, vals2)
    y = _dense_mid(h2, W1, n)
    g2 = _spmm_sc(y, rows2, cols2, vals2)
    return _combine_split(g2, n, n_user)


# speedup vs baseline: 1.5760x; 1.5760x over previous
import dataclasses
import functools

import jax
import jax.numpy as jnp
from jax import lax
from jax.experimental import pallas as pl
from jax.experimental.pallas import tpu as pltpu
from jax.experimental.pallas import tpu_sc as plsc

NC = 2
NS = 16
L = 16
NW = NC * NS
CHUNK = 40


def _dense_pre(user, item, w_user, w_item, w0):
    nu = user.shape[0]
    ni = item.shape[0]
    d = w0.shape[1]

    def body(u_ref, i_ref, wu_ref, wi_ref, w0_ref, o_ref):
        wu0 = jnp.dot(wu_ref[...], w0_ref[...], preferred_element_type=jnp.float32)
        wi0 = jnp.dot(wi_ref[...], w0_ref[...], preferred_element_type=jnp.float32)
        o_ref[:nu, :] = jnp.dot(u_ref[...], wu0, preferred_element_type=jnp.float32)
        o_ref[nu:, :] = jnp.dot(i_ref[...], wi0, preferred_element_type=jnp.float32)

    return pl.pallas_call(
        body,
        out_shape=jax.ShapeDtypeStruct((nu + ni, d), jnp.float32),
    )(user, item, w_user, w_item, w0)


def _dense_mid(h2, w1, n):
    npad = h2.shape[0] // 2
    d = w1.shape[1]

    def body(h_ref, w_ref, o_ref):
        h = jnp.maximum(h_ref[:n, :] + h_ref[npad : npad + n, :], 0.0)
        o_ref[...] = jnp.dot(h, w_ref[...], preferred_element_type=jnp.float32)

    return pl.pallas_call(
        body,
        out_shape=jax.ShapeDtypeStruct((n, d), jnp.float32),
    )(h2, w1)


def _combine_split(g2, n, n_user):
    npad = g2.shape[0] // 2
    d = g2.shape[1]

    def body(g_ref, u_ref, i_ref):
        u_ref[...] = g_ref[:n_user, :] + g_ref[npad : npad + n_user, :]
        i_ref[...] = g_ref[n_user:n, :] + g_ref[npad + n_user : npad + n, :]

    return pl.pallas_call(
        body,
        out_shape=(
            jax.ShapeDtypeStruct((n_user, d), jnp.float32),
            jax.ShapeDtypeStruct((n - n_user, d), jnp.float32),
        ),
    )(g2)


def _spmm_sc(x, rows2, cols2, vals2):
    n, d = x.shape
    nchunk = rows2.shape[0] // (NW * CHUNK)
    per_w = nchunk * CHUNK
    rpt = (-(-n // NS) + 7) // 8 * 8
    npad = rpt * NS

    mesh = plsc.VectorSubcoreMesh(core_axis_name="c", subcore_axis_name="s")
    cp = pltpu.CompilerParams()
    if "needs_layout_passes" in pltpu.CompilerParams.__dataclass_fields__:
        cp = dataclasses.replace(cp, needs_layout_passes=False)

    @functools.partial(
        pl.kernel,
        out_type=jax.ShapeDtypeStruct((NC * npad, d), jnp.float32),
        mesh=mesh,
        compiler_params=cp,
        scratch_types=[
            pltpu.VMEM_SHARED((npad, d), jnp.float32),
            pltpu.VMEM((nchunk * CHUNK,), jnp.int32),
            pltpu.VMEM((nchunk * CHUNK,), jnp.float32),
            pltpu.VMEM((CHUNK, d), jnp.float32),
            pltpu.VMEM((CHUNK, d), jnp.float32),
            pltpu.VMEM((CHUNK,), jnp.int32),
            pltpu.VMEM((CHUNK,), jnp.int32),
            pltpu.SemaphoreType.DMA,
            pltpu.SemaphoreType.DMA,
            pltpu.SemaphoreType.DMA,
        ],
    )
    def run(x_hbm, rows_hbm, cols_hbm, vals_hbm, zeros_hbm, out_hbm,
            acc, cols_loc, vals_loc, rbuf0, rbuf1, ridx0, ridx1,
            gsem0, gsem1, zsem):
        ci = lax.axis_index("c")
        si = lax.axis_index("s")
        wid = ci * NS + si
        rbase = wid * per_w
        zcopy = pltpu.async_copy(zeros_hbm.at[pl.ds(si * rpt, rpt)],
                                 acc.at[pl.ds(si * rpt, rpt)], zsem)
        pltpu.sync_copy(cols_hbm.at[wid], cols_loc)
        pltpu.sync_copy(vals_hbm.at[wid], vals_loc)
        zcopy.wait()
        plsc.subcore_barrier()

        def refs(q, rbuf, ridx, gsem):
            g = pltpu.make_async_copy(
                x_hbm.at[cols_loc.at[pl.ds(q * CHUNK, CHUNK)]], rbuf, gsem)
            r = pltpu.make_async_copy(
                rows_hbm.at[pl.ds(rbase + q * CHUNK, CHUNK)], ridx, gsem)
            return g, r

        def issue_gather(q, rbuf, ridx, gsem):
            pltpu.async_copy(
                x_hbm.at[cols_loc.at[pl.ds(q * CHUNK, CHUNK)]], rbuf, gsem)
            pltpu.async_copy(
                rows_hbm.at[pl.ds(rbase + q * CHUNK, CHUNK)], ridx, gsem)

        issue_gather(0, rbuf0, ridx0, gsem0)
        issue_gather(1, rbuf1, ridx1, gsem1)

        def half(q, rbuf, ridx, gsem):
            g, r = refs(q, rbuf, ridx, gsem)
            g.wait()
            r.wait()
            @pl.loop(0, CHUNK // L)
            def _grp(g):
                e0 = g * L
                for j in range(L):
                    idx = jnp.broadcast_to(q * CHUNK + e0 + j, (L,))
                    v = plsc.load_gather(vals_loc, [idx.astype(jnp.int32)])
                    for f in range(0, d, L):
                        rbuf[e0 + j, pl.ds(f, L)] = rbuf[e0 + j, pl.ds(f, L)] * v
            pltpu.sync_copy(rbuf, acc.at[ridx], add=True)

            @pl.when(q < nchunk - 2)
            def _():
                issue_gather(q + 2, rbuf, ridx, gsem)

        @pl.loop(0, nchunk, step=2)
        def _pair(k):
            half(k, rbuf0, ridx0, gsem0)
            half(k + 1, rbuf1, ridx1, gsem1)

        plsc.subcore_barrier()
        pltpu.sync_copy(acc.at[pl.ds(si * rpt, rpt)],
                        out_hbm.at[pl.ds(ci * npad + si * rpt, rpt)])

    zeros = jnp.zeros((npad, d), jnp.float32)
    return run(x, rows2, cols2, vals2, zeros)


def kernel(user_attrs_input, item_attrs_input, edge_index, adj_vals,
           W_user, W_item, W0, W1):
    n_user = user_attrs_input.shape[0]
    e = edge_index.shape[1]

    per_w = -(-e // (NW * 2 * CHUNK)) * 2 * CHUNK
    pad = NW * per_w - e
    rows = edge_index[0].astype(jnp.int32)
    cols = edge_index[1].astype(jnp.int32)
    vals = adj_vals.astype(jnp.float32)
    if pad:
        rows = jnp.concatenate([rows, jnp.zeros((pad,), jnp.int32)])
        cols = jnp.concatenate([cols, jnp.zeros((pad,), jnp.int32)])
        vals = jnp.concatenate([vals, jnp.zeros((pad,), jnp.float32)])
    rows2 = rows
    cols2 = cols.reshape(NW, per_w)
    vals2 = vals.reshape(NW, per_w)

    x0 = _dense_pre(user_attrs_input, item_attrs_input, W_user, W_item, W0)
    n = x0.shape[0]
    h2 = _spmm_sc(x0, rows2, cols2, vals2)
    y = _dense_mid(h2, W1, n)
    g2 = _spmm_sc(y, rows2, cols2, vals2)
    return _combine_split(g2, n, n_user)
